# baseline (speedup 1.0000x reference)
import functools

import jax
import jax.numpy as jnp
from jax.experimental import pallas as pl


def _chamfer_body(a1_ref, a2_ref, dist1_ref, dist2_ref):
    a1 = a1_ref[0]
    a2 = a2_ref[0]
    bn = a1.shape[1]
    m = a2.shape[1]
    x2 = jnp.sum(a1 * a1, axis=0, keepdims=True)
    y2 = jnp.sum(a2 * a2, axis=0, keepdims=True)
    ones_bn = jnp.ones((1, bn), dtype=a1.dtype)
    ones_m = jnp.ones((1, m), dtype=a2.dtype)
    a1aug = jnp.concatenate([a1, x2, ones_bn], axis=0)
    a2aug = jnp.concatenate([-2.0 * a2, ones_m, y2], axis=0)
    d = jax.lax.dot_general(
        a1aug, a2aug,
        dimension_numbers=(((0,), (0,)), ((), ())),
        preferred_element_type=jnp.float32,
    )
    dist1_ref[0, 0] = jnp.min(d, axis=1)
    colmin = jnp.min(d, axis=0)

    @pl.when(pl.program_id(1) == 0)
    def _init():
        dist2_ref[0, 0] = colmin

    @pl.when(pl.program_id(1) != 0)
    def _acc():
        dist2_ref[0, 0] = jnp.minimum(dist2_ref[0, 0], colmin)


@functools.partial(jax.jit, static_argnames=("block_n", "interpret"))
def _chamfer(array1, array2, block_n=512, interpret=False):
    b, n, _ = array1.shape
    m = array2.shape[1]
    a1t = array1.transpose(0, 2, 1)
    a2t = array2.transpose(0, 2, 1)
    nb = n // block_n
    grid = (b, nb)
    dist1, dist2 = pl.pallas_call(
        _chamfer_body,
        grid=grid,
        in_specs=[
            pl.BlockSpec((1, 3, block_n), lambda i, j: (i, 0, j)),
            pl.BlockSpec((1, 3, m), lambda i, j: (i, 0, 0)),
        ],
        out_specs=[
            pl.BlockSpec((1, 1, block_n), lambda i, j: (i * nb + j, 0, 0)),
            pl.BlockSpec((1, 1, m), lambda i, j: (i, 0, 0)),
        ],
        out_shape=[
            jax.ShapeDtypeStruct((b * nb, 1, block_n), jnp.float32),
            jax.ShapeDtypeStruct((b, 1, m), jnp.float32),
        ],
        interpret=interpret,
    )(a1t, a2t)
    return jnp.mean(dist1) + jnp.mean(dist2)


def kernel(array1, array2):
    return _chamfer(array1, array2)
